# baseline (speedup 1.0000x reference)
import functools

import jax
import jax.numpy as jnp
from jax import lax
from jax.experimental import pallas as pl
from jax.experimental.pallas import tpu as pltpu
from jax.experimental.pallas import tpu_sc as plsc

TH, TW = 15, 15
PAD_HT = (TH - 1) // 2
PAD_WL = (TW - 1) // 2
MARGIN = 0.25
GAMMA = 256.0
NEG = -1e30
PEN = -1e35

NW = 32
ROWS_PER_CHUNK = 8


def _sc_body(x_hbm, mw_hbm, out_hbm, xb, mb, lpb, lnb, stage, *,
             H, W, r0, r1, c0, c1):
    wid = lax.axis_index("s") * 2 + lax.axis_index("c")
    rows_w = H // NW
    n_chunks = rows_w // ROWS_PER_CHUNK
    ch_elems = ROWS_PER_CHUNK * W
    vecs_row = W // 16

    iota = lax.iota(jnp.int32, 16)
    sub4 = iota >> 2
    shift = (iota & 3) << 3
    penfirst = jnp.where(iota < c0, PEN, 0.0)
    penlast = jnp.where(iota >= 16 - (W - c1), PEN, 0.0)
    zeros16 = jnp.zeros((16,), jnp.float32)
    gdn = lax.GatherDimensionNumbers(
        offset_dims=(), collapsed_slice_dims=(0,), start_index_map=(0,))
    groups_row = W // 64

    def logits(v_base, mwvec, k, colpen, rowpen):
        x = xb[pl.ds(v_base, 16)]
        g = lax.gather(mwvec, ((4 * k) + sub4)[:, None], gdn,
                       slice_sizes=(1,),
                       mode=lax.GatherScatterMode.PROMISE_IN_BOUNDS)
        tm = ((g >> shift) & 1) != 0
        y = x * GAMMA
        lp = jnp.maximum(GAMMA * (1.0 + MARGIN) - y, 0.0) * ((1.0 - MARGIN) - x)
        ln = (jnp.maximum(y + GAMMA * MARGIN, 0.0) * (x - MARGIN)
              + (colpen + rowpen))
        lP = jnp.where(tm, lp, PEN)
        lN = jnp.where(tm, PEN, ln)
        return lP, lN

    def row_pass(r, carry, ch):
        Mp, Sp, Mn, Sn = carry
        g_row = wid * rows_w + ch * ROWS_PER_CHUNK + r
        rowpen = jnp.where((g_row >= r0) & (g_row < r1), 0.0, PEN)
        base = r * W

        def passA_group(grp, mx, colpens):
            gbase = base + grp * 64
            mwvec = mb[pl.ds(gbase >> 2, 16)]
            for k in range(4):
                mP, mN = mx
                lP, lN = logits(gbase + k * 16, mwvec, k,
                                colpens[k], rowpen)
                lpb[pl.ds(grp * 64 + k * 16, 16)] = lP
                lnb[pl.ds(grp * 64 + k * 16, 16)] = lN
                mx = (jnp.maximum(mP, lP), jnp.maximum(mN, lN))
            return mx

        nopen = (zeros16, zeros16, zeros16, zeros16)
        mx = (jnp.full((16,), NEG), jnp.full((16,), NEG))
        mx = passA_group(0, mx, (penfirst, zeros16, zeros16, zeros16))
        mx = lax.fori_loop(
            1, groups_row - 1,
            lambda grp, c: passA_group(grp, c, nopen), mx)
        mP, mN = passA_group(groups_row - 1, mx,
                             (zeros16, zeros16, zeros16, penlast))

        nMp = jnp.maximum(Mp, mP)
        nMn = jnp.maximum(Mn, mN)

        def passB_vec(v, s):
            sP, sN = s
            eP = jnp.exp(lpb[pl.ds(v * 16, 16)] - nMp)
            eN = jnp.exp(lnb[pl.ds(v * 16, 16)] - nMn)
            return sP + eP, sN + eN

        rsP, rsN = lax.fori_loop(0, vecs_row, passB_vec, (zeros16, zeros16))
        nSp = Sp * jnp.exp(Mp - nMp) + rsP
        nSn = Sn * jnp.exp(Mn - nMn) + rsN
        return nMp, nSp, nMn, nSn

    def chunk(ch, carry):
        off = pl.multiple_of(wid * rows_w * W + ch * ch_elems, 256)
        woff = pl.multiple_of(off >> 2, 64)
        pltpu.sync_copy(x_hbm.at[pl.ds(off, ch_elems)], xb)
        pltpu.sync_copy(mw_hbm.at[pl.ds(woff, ch_elems // 4)], mb)
        return lax.fori_loop(
            0, ROWS_PER_CHUNK, lambda r, c: row_pass(r, c, ch), carry)

    init = (jnp.full((16,), NEG), zeros16, jnp.full((16,), NEG), zeros16)
    Mp, Sp, Mn, Sn = lax.fori_loop(0, n_chunks, chunk, init)

    stage[0, :] = Mp
    stage[1, :] = Sp
    stage[2, :] = Mn
    stage[3, :] = Sn
    pltpu.sync_copy(stage, out_hbm.at[wid])


def _merge_body(p_ref, o_ref):
    p = p_ref[...]
    mp = p[:, 0, :]
    sp = p[:, 1, :]
    mn = p[:, 2, :]
    sn = p[:, 3, :]
    Mp = jnp.max(mp)
    Mn = jnp.max(mn)
    Sp = jnp.sum(sp * jnp.exp(mp - Mp))
    Sn = jnp.sum(sn * jnp.exp(mn - Mn))
    z = Mp + jnp.log(Sp) + Mn + jnp.log(Sn)
    o_ref[0, 0] = jnp.maximum(z, 0.0) + jnp.log1p(jnp.exp(-jnp.abs(z)))


def kernel(ypred, truthMask):
    B, H, W = ypred.shape
    mh, mw = truthMask.shape[-2], truthMask.shape[-1]
    r0 = PAD_HT - 1
    r1 = min(PAD_HT - TH + mh + 2, H)
    c0 = PAD_WL - 1
    c1 = min(PAD_WL - TW + mw + 2, W)

    x = ypred.reshape(H * W)
    mwords = jax.lax.bitcast_convert_type(
        truthMask.reshape(H * W // 4, 4).astype(jnp.uint8), jnp.int32)

    mesh = plsc.VectorSubcoreMesh(core_axis_name="c", subcore_axis_name="s")
    ch_elems = ROWS_PER_CHUNK * W

    sc = functools.partial(
        pl.kernel,
        out_type=jax.ShapeDtypeStruct((NW, 4, 16), jnp.float32),
        mesh=mesh,
        scratch_types=[
            pltpu.VMEM((ch_elems,), jnp.float32),
            pltpu.VMEM((ch_elems // 4,), jnp.int32),
            pltpu.VMEM((W,), jnp.float32),
            pltpu.VMEM((W,), jnp.float32),
            pltpu.VMEM((4, 16), jnp.float32),
        ],
    )(functools.partial(_sc_body, H=H, W=W, r0=r0, r1=r1, c0=c0, c1=c1))
    partials = sc(x, mwords)

    out = pl.pallas_call(
        _merge_body,
        out_specs=pl.BlockSpec(memory_space=pltpu.SMEM),
        out_shape=jax.ShapeDtypeStruct((1, 1), jnp.float32),
    )(partials)
    return out.reshape(B)


# speedup vs baseline: 1.0004x; 1.0004x over previous
import functools

import jax
import jax.numpy as jnp
from jax import lax
from jax.experimental import pallas as pl
from jax.experimental.pallas import tpu as pltpu
from jax.experimental.pallas import tpu_sc as plsc

TH, TW = 15, 15
PAD_HT = (TH - 1) // 2
PAD_WL = (TW - 1) // 2
MARGIN = 0.25
GAMMA = 256.0
NEG = -1e30
PEN = -1e35

NW = 32
ROWS_PER_CHUNK = 8


def _sc_body(x_hbm, mw_hbm, out_hbm, xb, mb, lpb, lnb, stage, *,
             H, W, r0, r1, c0, c1):
    wid = lax.axis_index("s") * 2 + lax.axis_index("c")
    rows_w = H // NW
    n_chunks = rows_w // ROWS_PER_CHUNK
    ch_elems = ROWS_PER_CHUNK * W
    vecs_row = W // 16

    iota = lax.iota(jnp.int32, 16)
    sub4 = iota >> 2
    shift = (iota & 3) << 3
    penfirst = jnp.where(iota < c0, PEN, 0.0)
    penlast = jnp.where(iota >= 16 - (W - c1), PEN, 0.0)
    zeros16 = jnp.zeros((16,), jnp.float32)
    gdn = lax.GatherDimensionNumbers(
        offset_dims=(), collapsed_slice_dims=(0,), start_index_map=(0,))
    groups_row = W // 64

    def logits(v_base, mwvec, k, colpen, rowpen):
        x = xb[pl.ds(v_base, 16)]
        g = lax.gather(mwvec, ((4 * k) + sub4)[:, None], gdn,
                       slice_sizes=(1,),
                       mode=lax.GatherScatterMode.PROMISE_IN_BOUNDS)
        tm = ((g >> shift) & 1) != 0
        y = x * GAMMA
        lp = jnp.maximum(GAMMA * (1.0 + MARGIN) - y, 0.0) * ((1.0 - MARGIN) - x)
        ln = (jnp.maximum(y + GAMMA * MARGIN, 0.0) * (x - MARGIN)
              + (colpen + rowpen))
        lP = jnp.where(tm, lp, PEN)
        lN = jnp.where(tm, PEN, ln)
        return lP, lN

    def row_pass(r, carry, ch):
        Mp, Sp, Mn, Sn = carry
        g_row = wid * rows_w + ch * ROWS_PER_CHUNK + r
        rowpen = jnp.where((g_row >= r0) & (g_row < r1), 0.0, PEN)
        base = r * W

        def passA_group(grp, mx, colpens):
            gbase = base + grp * 64
            mwvec = mb[pl.ds(gbase >> 2, 16)]
            for k in range(4):
                mP, mN = mx
                lP, lN = logits(gbase + k * 16, mwvec, k,
                                colpens[k], rowpen)
                lpb[pl.ds(grp * 64 + k * 16, 16)] = lP
                lnb[pl.ds(grp * 64 + k * 16, 16)] = lN
                mx = (jnp.maximum(mP, lP), jnp.maximum(mN, lN))
            return mx

        nopen = (zeros16, zeros16, zeros16, zeros16)
        mx = (jnp.full((16,), NEG), jnp.full((16,), NEG))
        mx = passA_group(0, mx, (penfirst, zeros16, zeros16, zeros16))
        mx = plsc.parallel_loop(1, groups_row - 1, unroll=2, carry=mx)(
            lambda grp, c: passA_group(grp, c, nopen))
        mP, mN = passA_group(groups_row - 1, mx,
                             (zeros16, zeros16, zeros16, penlast))

        nMp = jnp.maximum(Mp, mP)
        nMn = jnp.maximum(Mn, mN)

        def passB_vec(v, s):
            sP, sN = s
            eP = jnp.exp(lpb[pl.ds(v * 16, 16)] - nMp)
            eN = jnp.exp(lnb[pl.ds(v * 16, 16)] - nMn)
            return sP + eP, sN + eN

        rsP, rsN = plsc.parallel_loop(
            0, vecs_row, unroll=8, carry=(zeros16, zeros16))(passB_vec)
        nSp = Sp * jnp.exp(Mp - nMp) + rsP
        nSn = Sn * jnp.exp(Mn - nMn) + rsN
        return nMp, nSp, nMn, nSn

    def chunk(ch, carry):
        off = pl.multiple_of(wid * rows_w * W + ch * ch_elems, 256)
        woff = pl.multiple_of(off >> 2, 64)
        pltpu.sync_copy(x_hbm.at[pl.ds(off, ch_elems)], xb)
        pltpu.sync_copy(mw_hbm.at[pl.ds(woff, ch_elems // 4)], mb)
        return lax.fori_loop(
            0, ROWS_PER_CHUNK, lambda r, c: row_pass(r, c, ch), carry)

    init = (jnp.full((16,), NEG), zeros16, jnp.full((16,), NEG), zeros16)
    Mp, Sp, Mn, Sn = lax.fori_loop(0, n_chunks, chunk, init)

    stage[0, :] = Mp
    stage[1, :] = Sp
    stage[2, :] = Mn
    stage[3, :] = Sn
    pltpu.sync_copy(stage, out_hbm.at[wid])


def _merge_body(p_ref, o_ref):
    p = p_ref[...]
    mp = p[:, 0, :]
    sp = p[:, 1, :]
    mn = p[:, 2, :]
    sn = p[:, 3, :]
    Mp = jnp.max(mp)
    Mn = jnp.max(mn)
    Sp = jnp.sum(sp * jnp.exp(mp - Mp))
    Sn = jnp.sum(sn * jnp.exp(mn - Mn))
    z = Mp + jnp.log(Sp) + Mn + jnp.log(Sn)
    o_ref[0, 0] = jnp.maximum(z, 0.0) + jnp.log1p(jnp.exp(-jnp.abs(z)))


def kernel(ypred, truthMask):
    B, H, W = ypred.shape
    mh, mw = truthMask.shape[-2], truthMask.shape[-1]
    r0 = PAD_HT - 1
    r1 = min(PAD_HT - TH + mh + 2, H)
    c0 = PAD_WL - 1
    c1 = min(PAD_WL - TW + mw + 2, W)

    x = ypred.reshape(H * W)
    mwords = jax.lax.bitcast_convert_type(
        truthMask.reshape(H * W // 4, 4).astype(jnp.uint8), jnp.int32)

    mesh = plsc.VectorSubcoreMesh(core_axis_name="c", subcore_axis_name="s")
    ch_elems = ROWS_PER_CHUNK * W

    sc = functools.partial(
        pl.kernel,
        out_type=jax.ShapeDtypeStruct((NW, 4, 16), jnp.float32),
        mesh=mesh,
        scratch_types=[
            pltpu.VMEM((ch_elems,), jnp.float32),
            pltpu.VMEM((ch_elems // 4,), jnp.int32),
            pltpu.VMEM((W,), jnp.float32),
            pltpu.VMEM((W,), jnp.float32),
            pltpu.VMEM((4, 16), jnp.float32),
        ],
    )(functools.partial(_sc_body, H=H, W=W, r0=r0, r1=r1, c0=c0, c1=c1))
    partials = sc(x, mwords)

    out = pl.pallas_call(
        _merge_body,
        out_specs=pl.BlockSpec(memory_space=pltpu.SMEM),
        out_shape=jax.ShapeDtypeStruct((1, 1), jnp.float32),
    )(partials)
    return out.reshape(B)


# speedup vs baseline: 40.5324x; 40.5168x over previous
import functools

import jax
import jax.numpy as jnp
from jax.experimental import pallas as pl
from jax.experimental.pallas import tpu as pltpu

TH, TW = 15, 15
PAD_HT = (TH - 1) // 2
PAD_WL = (TW - 1) // 2
MARGIN = 0.25
GAMMA = 256.0
NEG = -1e30
PEN = -1e35


def _loss_body(x_ref, m_ref, o_ref, mp_a, sp_a, mn_a, sn_a, *,
               nrows, W, r0, r1, c0, c1):
    i = pl.program_id(0)
    nsub = nrows // 8

    @pl.when(i == 0)
    def _init():
        mp_a[...] = jnp.full((8, W), NEG, jnp.float32)
        sp_a[...] = jnp.zeros((8, W), jnp.float32)
        mn_a[...] = jnp.full((8, W), NEG, jnp.float32)
        sn_a[...] = jnp.zeros((8, W), jnp.float32)

    x = x_ref[...]
    tm = m_ref[...]

    rid = jax.lax.broadcasted_iota(jnp.int32, (nrows, 1), 0) + i * nrows
    rowpen = jnp.where((rid >= r0) & (rid < r1), 0.0, PEN)

    K = GAMMA * 1.4426950408889634
    y = x * K
    lp = jnp.maximum(K * (1.0 + MARGIN) - y, 0.0) * ((1.0 - MARGIN) - x)
    ln = jnp.maximum(y + K * MARGIN, 0.0) * (x - MARGIN) + rowpen
    lP = jnp.where(tm, lp, PEN).reshape(nsub, 8, W)
    lN = jnp.where(tm, PEN, ln).reshape(nsub, 8, W)

    mp_old = mp_a[...]
    mn_old = mn_a[...]
    mp = jnp.maximum(mp_old, jnp.max(lP, axis=0))
    mn = jnp.maximum(mn_old, jnp.max(lN, axis=0))

    ep = jnp.exp2(lP - mp[None, :, :])
    en = jnp.exp2(lN - mn[None, :, :])

    mp_a[...] = mp
    sp_a[...] = sp_a[...] * jnp.exp2(mp_old - mp) + jnp.sum(ep, axis=0)
    mn_a[...] = mn
    sn_a[...] = sn_a[...] * jnp.exp2(mn_old - mn) + jnp.sum(en, axis=0)

    @pl.when(i == pl.num_programs(0) - 1)
    def _fin():
        LN2 = 0.6931471805599453
        cid = jax.lax.broadcasted_iota(jnp.int32, (1, W), 1)
        col_ok = (cid >= c0) & (cid < c1)
        mp_c = mp_a[...]
        mn_c = jnp.where(col_ok, mn_a[...], NEG)
        sn_c = jnp.where(col_ok, sn_a[...], 0.0)
        Mp = jnp.max(mp_c)
        Mn = jnp.max(mn_c)
        Sp = jnp.sum(sp_a[...] * jnp.exp2(mp_c - Mp))
        Sn = jnp.sum(sn_c * jnp.exp2(mn_c - Mn))
        z = LN2 * (Mp + Mn) + jnp.log(Sp) + jnp.log(Sn)
        o_ref[0, 0] = jnp.maximum(z, 0.0) + jnp.log1p(jnp.exp(-jnp.abs(z)))


def kernel(ypred, truthMask):
    B, H, W = ypred.shape
    mh, mw = truthMask.shape[-2], truthMask.shape[-1]
    r0 = PAD_HT - 1
    r1 = min(PAD_HT - TH + mh + 2, H)
    c0 = PAD_WL - 1
    c1 = min(PAD_WL - TW + mw + 2, W)

    x = ypred.reshape(H, W)
    tm = truthMask.reshape(H, W)

    nrows = 256 if H % 256 == 0 else H
    grid = H // nrows

    out = pl.pallas_call(
        functools.partial(
            _loss_body, nrows=nrows, W=W, r0=r0, r1=r1, c0=c0, c1=c1
        ),
        grid=(grid,),
        in_specs=[
            pl.BlockSpec((nrows, W), lambda i: (i, 0)),
            pl.BlockSpec((nrows, W), lambda i: (i, 0)),
        ],
        out_specs=pl.BlockSpec(
            (1, 1), lambda i: (0, 0), memory_space=pltpu.SMEM
        ),
        out_shape=jax.ShapeDtypeStruct((1, 1), jnp.float32),
        scratch_shapes=[
            pltpu.VMEM((8, W), jnp.float32),
            pltpu.VMEM((8, W), jnp.float32),
            pltpu.VMEM((8, W), jnp.float32),
            pltpu.VMEM((8, W), jnp.float32),
        ],
        compiler_params=pltpu.CompilerParams(
            dimension_semantics=("arbitrary",),
        ),
    )(x, tm)
    return out.reshape(B)
